# baseline (speedup 1.0000x reference)
import functools

import jax
import jax.numpy as jnp
from jax import lax
from jax.experimental import pallas as pl
from jax.experimental.pallas import tpu as pltpu
from jax.experimental.pallas import tpu_sc as plsc

_NU = 5000
_NI = 5000
_NA = 5000
_D = 128
_B = 4096
_ALPHA = 0.1
_MARGIN = 0.2

_N = 10000
_NPAD = 10240
_E = 320000
_EPAD = 327680
_EPW = _EPAD // 32
_CK = 64
_NCH = _EPW // _CK
_TOTCH = _EPAD // _CK
_CH0 = 80
_CH1 = (_TOTCH - 16 * _CH0) // 16
_IBK = 16
_NBUF = 4
_CKD = 128
_NCHD = _EPW // _CKD
_W16 = 16
_NC, _NS = 2, 16
_SL = _NPAD // _NS

_BR = 1280
_GRID = _NPAD // _BR
_BRL = 512


def _mesh():
    return plsc.VectorSubcoreMesh(core_axis_name="c", subcore_axis_name="s")


@functools.partial(
    pl.kernel,
    out_type=jax.ShapeDtypeStruct((3, _NC, _NPAD, _W16), jnp.float32),
    mesh=_mesh(),
    scratch_types=[
        pltpu.VMEM((_NCHD, _CKD), jnp.int32),
        pltpu.VMEM((_CKD, _W16), jnp.float32),
        pltpu.VMEM_SHARED((_NPAD, _W16), jnp.float32),
        pltpu.SemaphoreType.DMA,
        pltpu.SemaphoreType.DMA,
        pltpu.SemaphoreType.DMA,
        pltpu.SemaphoreType.DMA,
    ],
)
def _degk(r3_ui, r3_ia, r3_ua, z16, onerows, out, ivd, ones_v, acc,
          dsem0, dsem1, dsem2, dsem3):
    c = lax.axis_index("c")
    s = lax.axis_index("s")
    w = c * _NS + s
    sems = (dsem0, dsem1, dsem2, dsem3)
    pltpu.sync_copy(onerows, ones_v)
    for g, rr in enumerate((r3_ui, r3_ia, r3_ua)):
        pltpu.sync_copy(z16, acc.at[pl.ds(s * _SL, _SL)])
        pltpu.sync_copy(rr.at[w], ivd)
        plsc.subcore_barrier()

        @pl.loop(0, _NCHD, step=4)
        def _chunk(k):
            for b in range(4):
                pltpu.async_copy(ones_v, acc.at[ivd.at[k + b]], sems[b],
                                 add=True)
            for b in range(4):
                pltpu.make_async_copy(ones_v, acc.at[ivd.at[k + b]],
                                      sems[b]).wait()

        plsc.subcore_barrier()
        pltpu.sync_copy(acc.at[pl.ds(s * _SL, _SL)],
                        out.at[g, c, pl.ds(s * _SL, _SL)])
        plsc.subcore_barrier()


@functools.partial(
    pl.kernel,
    out_type=jax.ShapeDtypeStruct((_NC, _NPAD, _D), jnp.float32),
    mesh=_mesh(),
    scratch_types=[
        pltpu.VMEM((_IBK, 2, _CK), jnp.int32),
        pltpu.VMEM((_NBUF, _CK, _D), jnp.float32),
        pltpu.VMEM_SHARED((_NPAD, _D), jnp.float32),
        pltpu.SemaphoreType.DMA,
        pltpu.SemaphoreType.DMA,
        pltpu.SemaphoreType.DMA,
        pltpu.SemaphoreType.DMA,
        pltpu.SemaphoreType.DMA,
        pltpu.SemaphoreType.DMA,
        pltpu.SemaphoreType.DMA,
        pltpu.SemaphoreType.DMA,
    ],
)
def _adjsum(table, rc3, zrows, out, iv, rbuf, acc,
            g0, g1, g2, g3, s0, s1, s2, s3):
    c = lax.axis_index("c")
    s = lax.axis_index("s")
    w = c * _NS + s
    pltpu.sync_copy(zrows, acc.at[pl.ds(s * _SL, _SL)])
    plsc.subcore_barrier()
    gsems = (g0, g1, g2, g3)
    ssems = (s0, s1, s2, s3)

    def gstart(k, b):
        pltpu.async_copy(table.at[iv.at[k, 0]], rbuf.at[b], gsems[b])

    def gwait(k, b):
        pltpu.make_async_copy(table.at[iv.at[k, 0]], rbuf.at[b],
                              gsems[b]).wait()

    def sstart(k, b):
        pltpu.async_copy(rbuf.at[b], acc.at[iv.at[k, 1]], ssems[b], add=True)

    def swait(k, b):
        pltpu.make_async_copy(rbuf.at[b], acc.at[iv.at[k, 1]],
                              ssems[b]).wait()

    def ring(chunk_base, nch):
        @pl.loop(0, nch // _IBK)
        def _blk(j):
            pltpu.sync_copy(rc3.at[pl.ds(chunk_base + j * _IBK, _IBK)], iv)
            for b in range(_NBUF):
                gstart(b, b)

            @pl.loop(0, _IBK, step=_NBUF)
            def _grp(k):
                for b in range(_NBUF):
                    gwait(k + b, b)
                    sstart(k + b, b)
                for b in range(_NBUF):
                    @pl.when(k + _NBUF + b < _IBK)
                    def _(b=b):
                        swait(k + b, b)
                        gstart(k + _NBUF + b, b)

            for b in range(_NBUF):
                swait(_IBK - _NBUF + b, b)

    @pl.when(c == 0)
    def _():
        ring(s * _CH0, _CH0)

    @pl.when(c == 1)
    def _():
        ring(16 * _CH0 + s * _CH1, _CH1)

    plsc.subcore_barrier()
    pltpu.sync_copy(acc.at[pl.ds(s * _SL, _SL)],
                    out.at[c, pl.ds(s * _SL, _SL)])


@functools.partial(
    pl.kernel,
    out_type=jax.ShapeDtypeStruct((5, 2 * _B, _D), jnp.float32),
    mesh=_mesh(),
    scratch_types=[
        pltpu.VMEM((128,), jnp.int32),
        pltpu.VMEM((128, _D), jnp.float32),
        pltpu.SemaphoreType.DMA,
    ],
)
def _gather5(ui_rep, ua_rep, ia_rep, hat, utd, itd, itmd, out, iv, rbuf, sem):
    c = lax.axis_index("c")
    s = lax.axis_index("s")
    w = c * _NS + s
    rpw = (2 * _B) // 32
    tabs = (ui_rep, ui_rep, ua_rep, ia_rep, hat)
    idxs = (utd, itd, utd, itmd, utd)
    for t in range(5):
        for j in range(rpw // 128):
            base = w * rpw + j * 128
            pltpu.sync_copy(idxs[t].at[pl.ds(base, 128)], iv)
            pltpu.async_copy(tabs[t].at[iv], rbuf, sem).wait()
            pltpu.sync_copy(rbuf, out.at[t, pl.ds(base, 128)])


def _disp_body(dp, idp, iap, uap, disb, rcpb, xs):
    d = dp[...]
    deg16 = d[:, 0] + d[:, 1]
    deg = jnp.broadcast_to(deg16[:, :, 0:1], (3, _BR, _D))
    dis = jnp.where(deg > 0.0, lax.rsqrt(deg), 0.0)
    disb[...] = dis
    rcpb[...] = 1.0 / jnp.maximum(deg[2], 1.0)
    xs[0] = dis[0] * idp[...]
    xs[1] = dis[1] * iap[...]
    xs[2] = dis[2] * uap[...]


_disp = pl.pallas_call(
    _disp_body,
    grid=(_GRID,),
    in_specs=[
        pl.BlockSpec((3, _NC, _BR, _W16), lambda i: (0, 0, i, 0)),
        pl.BlockSpec((_BR, _D), lambda i: (i, 0)),
        pl.BlockSpec((_BR, _D), lambda i: (i, 0)),
        pl.BlockSpec((_BR, _D), lambda i: (i, 0)),
    ],
    out_specs=[
        pl.BlockSpec((3, _BR, _D), lambda i: (0, i, 0)),
        pl.BlockSpec((_BR, _D), lambda i: (i, 0)),
        pl.BlockSpec((3, _BR, _D), lambda i: (0, i, 0)),
    ],
    out_shape=[
        jax.ShapeDtypeStruct((3, _NPAD, _D), jnp.float32),
        jax.ShapeDtypeStruct((_NPAD, _D), jnp.float32),
        jax.ShapeDtypeStruct((3, _NPAD, _D), jnp.float32),
    ],
)


def _merge_specs(n_in, n_out):
    ins = [pl.BlockSpec((_NC, _BR, _D), lambda i: (0, i, 0))]
    ins += [pl.BlockSpec((_BR, _D), lambda i: (i, 0)) for _ in range(n_in - 1)]
    outs = [pl.BlockSpec((_BR, _D), lambda i: (i, 0)) for _ in range(n_out)]
    shp = [jax.ShapeDtypeStruct((_NPAD, _D), jnp.float32) for _ in range(n_out)]
    return dict(grid=(_GRID,), in_specs=ins, out_specs=outs, out_shape=shp)


def _merge_mid_body(p, db, b, accn, xsn):
    t = db[...] * (p[0] + p[1])
    accn[...] = b[...] + t
    xsn[...] = db[...] * t


_merge_mid = pl.pallas_call(_merge_mid_body, **_merge_specs(3, 2))


def _make_merge_fin(scale):
    def body(p, db, b, rep):
        rep[...] = (b[...] + db[...] * (p[0] + p[1])) * scale
    return pl.pallas_call(body, **_merge_specs(3, 1))


_merge_fin4 = _make_merge_fin(0.25)
_merge_fin2 = _make_merge_fin(0.5)


def _merge_mean_body(p, rcp, hat):
    hat[...] = rcp[...] * (p[0] + p[1])


_merge_mean = pl.pallas_call(_merge_mean_body, **_merge_specs(2, 1))


def _loss_body(p_ref, n_ref, out_ref, acc_ref):
    i = pl.program_id(0)
    p = p_ref[...]
    n = n_ref[...]

    def dots(x):
        cf = jnp.sum(x[0] * x[1], axis=-1, keepdims=True)
        kg = jnp.sum(x[2] * x[3], axis=-1, keepdims=True)
        ha = jnp.sum(x[4] * x[3], axis=-1, keepdims=True)
        return cf, kg, ha

    cfp, kgp, hap = dots(p)
    cfn, kgn, han = dots(n)
    sp_ = jax.nn.sigmoid(hap)
    sn_ = jax.nn.sigmoid(han)
    z = (cfn + kgn * sn_) - (cfp + kgp * sp_)
    l1 = jnp.maximum(z, 0.0) + jnp.log(1.0 + jnp.exp(-jnp.abs(z)))
    l2 = jnp.maximum(sp_ - sn_ - _MARGIN, 0.0)

    @pl.when(i == 0)
    def _():
        acc_ref[0] = 0.0
        acc_ref[1] = 0.0

    acc_ref[0] += jnp.sum(l1)
    acc_ref[1] += jnp.sum(l2)

    @pl.when(i == pl.num_programs(0) - 1)
    def _():
        out_ref[...] = jnp.full((1, 1),
                                acc_ref[0] / _B + _ALPHA * (acc_ref[1] / _B))


_loss_call = pl.pallas_call(
    _loss_body,
    grid=(_B // _BRL,),
    in_specs=[
        pl.BlockSpec((5, _BRL, _D), lambda i: (0, i, 0)),
        pl.BlockSpec((5, _BRL, _D), lambda i: (0, i + _B // _BRL, 0)),
    ],
    out_specs=pl.BlockSpec((1, 1), lambda i: (0, 0)),
    out_shape=jax.ShapeDtypeStruct((1, 1), jnp.float32),
    scratch_shapes=[pltpu.SMEM((2,), jnp.float32)],
)


def _reg_body(idr, upr, ipr, out):
    out[...] = jnp.full((1, 1), jnp.sum(idr[...] ** 2) / (_N * _D)
                        + jnp.sum(upr[...] ** 2) / (_NU * _D)
                        + jnp.sum(ipr[...] ** 2) / (_NI * _D))


_reg_call = pl.pallas_call(
    _reg_body,
    grid=(1,),
    in_specs=[
        pl.BlockSpec((_N, _D), lambda i: (0, 0)),
        pl.BlockSpec((_NU, _D), lambda i: (0, 0)),
        pl.BlockSpec((_NI, _D), lambda i: (0, 0)),
    ],
    out_specs=pl.BlockSpec((1, 1), lambda i: (0, 0)),
    out_shape=jax.ShapeDtypeStruct((1, 1), jnp.float32),
)


def _pad_edges(ei):
    ei = ei.astype(jnp.int32)
    pad = jnp.full((_EPAD - _E,), _N, jnp.int32)
    rows = jnp.concatenate([ei[0], pad]).reshape(_TOTCH, 1, _CK)
    cols = jnp.concatenate([ei[1], pad]).reshape(_TOTCH, 1, _CK)
    rc3 = jnp.concatenate([rows, cols], axis=1)
    return rc3, rows.reshape(32, _NCHD, _CKD)


def kernel(id_embedding, user_pre, item_pre, attribute, user_tensor,
           item_tensor, ui_edge_index, ia_edge_index, ua_edge_index):
    rowpad = ((0, _NPAD - _N), (0, 0))
    idp = jnp.pad(id_embedding, rowpad)
    ia0p = jnp.pad(jnp.concatenate([item_pre, attribute], axis=0), rowpad)
    ua0p = jnp.pad(jnp.concatenate([user_pre, attribute], axis=0), rowpad)
    rc_ui, r3_ui = _pad_edges(ui_edge_index)
    rc_ia, r3_ia = _pad_edges(ia_edge_index)
    rc_ua, r3_ua = _pad_edges(ua_edge_index)
    zrows = jnp.zeros((_SL, _D), jnp.float32)
    z16 = jnp.zeros((_SL, _W16), jnp.float32)
    onerows = jnp.ones((_CKD, _W16), jnp.float32)

    degp = _degk(r3_ui, r3_ia, r3_ua, z16, onerows)
    disb, rcpb, xs0 = _disp(degp, idp, ia0p, ua0p)

    p = _adjsum(xs0[0], rc_ui, zrows)
    acc1, xs = _merge_mid(p, disb[0], idp)
    p = _adjsum(xs, rc_ui, zrows)
    acc2, xs = _merge_mid(p, disb[0], acc1)
    p = _adjsum(xs, rc_ui, zrows)
    ui_rep, = _merge_fin4(p, disb[0], acc2)

    p = _adjsum(xs0[1], rc_ia, zrows)
    ia_rep, = _merge_fin2(p, disb[1], ia0p)

    p = _adjsum(xs0[2], rc_ua, zrows)
    ua_rep, = _merge_fin2(p, disb[2], ua0p)

    p = _adjsum(ua0p, rc_ua, zrows)
    hat, = _merge_mean(p, rcpb)

    ut = user_tensor.astype(jnp.int32)
    it = item_tensor.astype(jnp.int32)
    utd = jnp.concatenate([ut[:, 0], ut[:, 1]])
    itd = jnp.concatenate([it[:, 0], it[:, 1]])

    g5 = _gather5(ui_rep, ua_rep, ia_rep, hat, utd, itd, itd - _NU)
    loss = _loss_call(g5, g5)
    reg = _reg_call(id_embedding, user_pre, item_pre)
    return loss[0, 0], reg[0, 0]


# speedup vs baseline: 1.0930x; 1.0930x over previous
import functools

import jax
import jax.numpy as jnp
from jax import lax
from jax.experimental import pallas as pl
from jax.experimental.pallas import tpu as pltpu
from jax.experimental.pallas import tpu_sc as plsc

_NU = 5000
_NI = 5000
_NA = 5000
_D = 128
_B = 4096
_ALPHA = 0.1
_MARGIN = 0.2

_N = 10000
_NPAD = 10240
_E = 320000
_EPAD = 327680
_EPW = _EPAD // 32
_CK = 64
_NCH = _EPW // _CK
_TOTCH = _EPAD // _CK
_CH0 = 192
_CH1 = (_TOTCH - 16 * _CH0) // 16
_IBK = 16
_NBUF = 4
_CKD = 128
_NCHD = _EPW // _CKD
_W16 = 16
_NC, _NS = 2, 16
_SL = _NPAD // _NS

_BR = 1280
_GRID = _NPAD // _BR
_BRL = 512


def _mesh():
    return plsc.VectorSubcoreMesh(core_axis_name="c", subcore_axis_name="s")


@functools.partial(
    pl.kernel,
    out_type=jax.ShapeDtypeStruct((3, _NC, _NPAD, _W16), jnp.float32),
    mesh=_mesh(),
    scratch_types=[
        pltpu.VMEM((_NCHD, _CKD), jnp.int32),
        pltpu.VMEM((_CKD, _W16), jnp.float32),
        pltpu.VMEM_SHARED((_NPAD, _W16), jnp.float32),
        pltpu.SemaphoreType.DMA,
        pltpu.SemaphoreType.DMA,
        pltpu.SemaphoreType.DMA,
        pltpu.SemaphoreType.DMA,
    ],
)
def _degk(r3_ui, r3_ia, r3_ua, z16, onerows, out, ivd, ones_v, acc,
          dsem0, dsem1, dsem2, dsem3):
    c = lax.axis_index("c")
    s = lax.axis_index("s")
    w = c * _NS + s
    sems = (dsem0, dsem1, dsem2, dsem3)
    pltpu.sync_copy(onerows, ones_v)
    for g, rr in enumerate((r3_ui, r3_ia, r3_ua)):
        pltpu.sync_copy(z16, acc.at[pl.ds(s * _SL, _SL)])
        pltpu.sync_copy(rr.at[w], ivd)
        plsc.subcore_barrier()

        @pl.loop(0, _NCHD, step=4)
        def _chunk(k):
            for b in range(4):
                pltpu.async_copy(ones_v, acc.at[ivd.at[k + b]], sems[b],
                                 add=True)
            for b in range(4):
                pltpu.make_async_copy(ones_v, acc.at[ivd.at[k + b]],
                                      sems[b]).wait()

        plsc.subcore_barrier()
        pltpu.sync_copy(acc.at[pl.ds(s * _SL, _SL)],
                        out.at[g, c, pl.ds(s * _SL, _SL)])
        plsc.subcore_barrier()


@functools.partial(
    pl.kernel,
    out_type=jax.ShapeDtypeStruct((_NC, _NPAD, _D), jnp.float32),
    mesh=_mesh(),
    scratch_types=[
        pltpu.VMEM((_IBK, 2, _CK), jnp.int32),
        pltpu.VMEM((_NBUF, _CK, _D), jnp.float32),
        pltpu.VMEM_SHARED((_NPAD, _D), jnp.float32),
        pltpu.SemaphoreType.DMA,
        pltpu.SemaphoreType.DMA,
        pltpu.SemaphoreType.DMA,
        pltpu.SemaphoreType.DMA,
        pltpu.SemaphoreType.DMA,
        pltpu.SemaphoreType.DMA,
        pltpu.SemaphoreType.DMA,
        pltpu.SemaphoreType.DMA,
    ],
)
def _adjsum(table, rc3, zrows, out, iv, rbuf, acc,
            g0, g1, g2, g3, s0, s1, s2, s3):
    c = lax.axis_index("c")
    s = lax.axis_index("s")
    w = c * _NS + s
    pltpu.sync_copy(zrows, acc.at[pl.ds(s * _SL, _SL)])
    plsc.subcore_barrier()
    gsems = (g0, g1, g2, g3)
    ssems = (s0, s1, s2, s3)

    def gstart(k, b):
        pltpu.async_copy(table.at[iv.at[k, 0]], rbuf.at[b], gsems[b])

    def gwait(k, b):
        pltpu.make_async_copy(table.at[iv.at[k, 0]], rbuf.at[b],
                              gsems[b]).wait()

    def sstart(k, b):
        pltpu.async_copy(rbuf.at[b], acc.at[iv.at[k, 1]], ssems[b], add=True)

    def swait(k, b):
        pltpu.make_async_copy(rbuf.at[b], acc.at[iv.at[k, 1]],
                              ssems[b]).wait()

    def ring(chunk_base, nch):
        @pl.loop(0, nch // _IBK)
        def _blk(j):
            pltpu.sync_copy(rc3.at[pl.ds(chunk_base + j * _IBK, _IBK)], iv)
            for b in range(_NBUF):
                gstart(b, b)

            @pl.loop(0, _IBK, step=_NBUF)
            def _grp(k):
                for b in range(_NBUF):
                    gwait(k + b, b)
                    sstart(k + b, b)
                for b in range(_NBUF):
                    @pl.when(k + _NBUF + b < _IBK)
                    def _(b=b):
                        swait(k + b, b)
                        gstart(k + _NBUF + b, b)

            for b in range(_NBUF):
                swait(_IBK - _NBUF + b, b)

    @pl.when(c == 0)
    def _():
        ring(s * _CH0, _CH0)

    @pl.when(c == 1)
    def _():
        ring(16 * _CH0 + s * _CH1, _CH1)

    plsc.subcore_barrier()
    pltpu.sync_copy(acc.at[pl.ds(s * _SL, _SL)],
                    out.at[c, pl.ds(s * _SL, _SL)])


@functools.partial(
    pl.kernel,
    out_type=jax.ShapeDtypeStruct((5, 2 * _B, _D), jnp.float32),
    mesh=_mesh(),
    scratch_types=[
        pltpu.VMEM((128,), jnp.int32),
        pltpu.VMEM((128, _D), jnp.float32),
        pltpu.SemaphoreType.DMA,
    ],
)
def _gather5(ui_rep, ua_rep, ia_rep, hat, utd, itd, itmd, out, iv, rbuf, sem):
    c = lax.axis_index("c")
    s = lax.axis_index("s")
    w = c * _NS + s
    rpw = (2 * _B) // 32
    tabs = (ui_rep, ui_rep, ua_rep, ia_rep, hat)
    idxs = (utd, itd, utd, itmd, utd)
    for t in range(5):
        for j in range(rpw // 128):
            base = w * rpw + j * 128
            pltpu.sync_copy(idxs[t].at[pl.ds(base, 128)], iv)
            pltpu.async_copy(tabs[t].at[iv], rbuf, sem).wait()
            pltpu.sync_copy(rbuf, out.at[t, pl.ds(base, 128)])


def _disp_body(dp, idp, iap, uap, disb, rcpb, xs):
    d = dp[...]
    deg16 = d[:, 0] + d[:, 1]
    deg = jnp.broadcast_to(deg16[:, :, 0:1], (3, _BR, _D))
    dis = jnp.where(deg > 0.0, lax.rsqrt(deg), 0.0)
    disb[...] = dis
    rcpb[...] = 1.0 / jnp.maximum(deg[2], 1.0)
    xs[0] = dis[0] * idp[...]
    xs[1] = dis[1] * iap[...]
    xs[2] = dis[2] * uap[...]


_disp = pl.pallas_call(
    _disp_body,
    grid=(_GRID,),
    in_specs=[
        pl.BlockSpec((3, _NC, _BR, _W16), lambda i: (0, 0, i, 0)),
        pl.BlockSpec((_BR, _D), lambda i: (i, 0)),
        pl.BlockSpec((_BR, _D), lambda i: (i, 0)),
        pl.BlockSpec((_BR, _D), lambda i: (i, 0)),
    ],
    out_specs=[
        pl.BlockSpec((3, _BR, _D), lambda i: (0, i, 0)),
        pl.BlockSpec((_BR, _D), lambda i: (i, 0)),
        pl.BlockSpec((3, _BR, _D), lambda i: (0, i, 0)),
    ],
    out_shape=[
        jax.ShapeDtypeStruct((3, _NPAD, _D), jnp.float32),
        jax.ShapeDtypeStruct((_NPAD, _D), jnp.float32),
        jax.ShapeDtypeStruct((3, _NPAD, _D), jnp.float32),
    ],
)


def _merge_specs(n_in, n_out):
    ins = [pl.BlockSpec((_NC, _BR, _D), lambda i: (0, i, 0))]
    ins += [pl.BlockSpec((_BR, _D), lambda i: (i, 0)) for _ in range(n_in - 1)]
    outs = [pl.BlockSpec((_BR, _D), lambda i: (i, 0)) for _ in range(n_out)]
    shp = [jax.ShapeDtypeStruct((_NPAD, _D), jnp.float32) for _ in range(n_out)]
    return dict(grid=(_GRID,), in_specs=ins, out_specs=outs, out_shape=shp)


def _merge_mid_body(p, db, b, accn, xsn):
    t = db[...] * (p[0] + p[1])
    accn[...] = b[...] + t
    xsn[...] = db[...] * t


_merge_mid = pl.pallas_call(_merge_mid_body, **_merge_specs(3, 2))


def _make_merge_fin(scale):
    def body(p, db, b, rep):
        rep[...] = (b[...] + db[...] * (p[0] + p[1])) * scale
    return pl.pallas_call(body, **_merge_specs(3, 1))


_merge_fin4 = _make_merge_fin(0.25)
_merge_fin2 = _make_merge_fin(0.5)


def _merge_mean_body(p, rcp, hat):
    hat[...] = rcp[...] * (p[0] + p[1])


_merge_mean = pl.pallas_call(_merge_mean_body, **_merge_specs(2, 1))


def _loss_body(p_ref, n_ref, out_ref, acc_ref):
    i = pl.program_id(0)
    p = p_ref[...]
    n = n_ref[...]

    def dots(x):
        cf = jnp.sum(x[0] * x[1], axis=-1, keepdims=True)
        kg = jnp.sum(x[2] * x[3], axis=-1, keepdims=True)
        ha = jnp.sum(x[4] * x[3], axis=-1, keepdims=True)
        return cf, kg, ha

    cfp, kgp, hap = dots(p)
    cfn, kgn, han = dots(n)
    sp_ = jax.nn.sigmoid(hap)
    sn_ = jax.nn.sigmoid(han)
    z = (cfn + kgn * sn_) - (cfp + kgp * sp_)
    l1 = jnp.maximum(z, 0.0) + jnp.log(1.0 + jnp.exp(-jnp.abs(z)))
    l2 = jnp.maximum(sp_ - sn_ - _MARGIN, 0.0)

    @pl.when(i == 0)
    def _():
        acc_ref[0] = 0.0
        acc_ref[1] = 0.0

    acc_ref[0] += jnp.sum(l1)
    acc_ref[1] += jnp.sum(l2)

    @pl.when(i == pl.num_programs(0) - 1)
    def _():
        out_ref[...] = jnp.full((1, 1),
                                acc_ref[0] / _B + _ALPHA * (acc_ref[1] / _B))


_loss_call = pl.pallas_call(
    _loss_body,
    grid=(_B // _BRL,),
    in_specs=[
        pl.BlockSpec((5, _BRL, _D), lambda i: (0, i, 0)),
        pl.BlockSpec((5, _BRL, _D), lambda i: (0, i + _B // _BRL, 0)),
    ],
    out_specs=pl.BlockSpec((1, 1), lambda i: (0, 0)),
    out_shape=jax.ShapeDtypeStruct((1, 1), jnp.float32),
    scratch_shapes=[pltpu.SMEM((2,), jnp.float32)],
)


def _reg_body(idr, upr, ipr, out):
    out[...] = jnp.full((1, 1), jnp.sum(idr[...] ** 2) / (_N * _D)
                        + jnp.sum(upr[...] ** 2) / (_NU * _D)
                        + jnp.sum(ipr[...] ** 2) / (_NI * _D))


_reg_call = pl.pallas_call(
    _reg_body,
    grid=(1,),
    in_specs=[
        pl.BlockSpec((_N, _D), lambda i: (0, 0)),
        pl.BlockSpec((_NU, _D), lambda i: (0, 0)),
        pl.BlockSpec((_NI, _D), lambda i: (0, 0)),
    ],
    out_specs=pl.BlockSpec((1, 1), lambda i: (0, 0)),
    out_shape=jax.ShapeDtypeStruct((1, 1), jnp.float32),
)


def _pad_edges(ei):
    ei = ei.astype(jnp.int32)
    pad = jnp.full((_EPAD - _E,), _N, jnp.int32)
    rows = jnp.concatenate([ei[0], pad]).reshape(_TOTCH, 1, _CK)
    cols = jnp.concatenate([ei[1], pad]).reshape(_TOTCH, 1, _CK)
    rc3 = jnp.concatenate([rows, cols], axis=1)
    return rc3, rows.reshape(32, _NCHD, _CKD)


def kernel(id_embedding, user_pre, item_pre, attribute, user_tensor,
           item_tensor, ui_edge_index, ia_edge_index, ua_edge_index):
    rowpad = ((0, _NPAD - _N), (0, 0))
    idp = jnp.pad(id_embedding, rowpad)
    ia0p = jnp.pad(jnp.concatenate([item_pre, attribute], axis=0), rowpad)
    ua0p = jnp.pad(jnp.concatenate([user_pre, attribute], axis=0), rowpad)
    rc_ui, r3_ui = _pad_edges(ui_edge_index)
    rc_ia, r3_ia = _pad_edges(ia_edge_index)
    rc_ua, r3_ua = _pad_edges(ua_edge_index)
    zrows = jnp.zeros((_SL, _D), jnp.float32)
    z16 = jnp.zeros((_SL, _W16), jnp.float32)
    onerows = jnp.ones((_CKD, _W16), jnp.float32)

    degp = _degk(r3_ui, r3_ia, r3_ua, z16, onerows)
    disb, rcpb, xs0 = _disp(degp, idp, ia0p, ua0p)

    p = _adjsum(xs0[0], rc_ui, zrows)
    acc1, xs = _merge_mid(p, disb[0], idp)
    p = _adjsum(xs, rc_ui, zrows)
    acc2, xs = _merge_mid(p, disb[0], acc1)
    p = _adjsum(xs, rc_ui, zrows)
    ui_rep, = _merge_fin4(p, disb[0], acc2)

    p = _adjsum(xs0[1], rc_ia, zrows)
    ia_rep, = _merge_fin2(p, disb[1], ia0p)

    p = _adjsum(xs0[2], rc_ua, zrows)
    ua_rep, = _merge_fin2(p, disb[2], ua0p)

    p = _adjsum(ua0p, rc_ua, zrows)
    hat, = _merge_mean(p, rcpb)

    ut = user_tensor.astype(jnp.int32)
    it = item_tensor.astype(jnp.int32)
    utd = jnp.concatenate([ut[:, 0], ut[:, 1]])
    itd = jnp.concatenate([it[:, 0], it[:, 1]])

    g5 = _gather5(ui_rep, ua_rep, ia_rep, hat, utd, itd, itd - _NU)
    loss = _loss_call(g5, g5)
    reg = _reg_call(id_embedding, user_pre, item_pre)
    return loss[0, 0], reg[0, 0]


# speedup vs baseline: 1.1655x; 1.0663x over previous
import functools

import jax
import jax.numpy as jnp
from jax import lax
from jax.experimental import pallas as pl
from jax.experimental.pallas import tpu as pltpu
from jax.experimental.pallas import tpu_sc as plsc

_NU = 5000
_NI = 5000
_NA = 5000
_D = 128
_B = 4096
_ALPHA = 0.1
_MARGIN = 0.2

_N = 10000
_NPAD = 10240
_E = 320000
_EPAD = 327680
_EPW = _EPAD // 32
_CK = 64
_NCH = _EPW // _CK
_TOTCH = _EPAD // _CK
_CH0 = 256
_CH1 = (_TOTCH - 16 * _CH0) // 16
_IBK = 16
_NBUF = 4
_CKD = 128
_NCHD = _EPW // _CKD
_W16 = 16
_NC, _NS = 2, 16
_SL = _NPAD // _NS

_BR = 1280
_GRID = _NPAD // _BR
_BRL = 512


def _mesh():
    return plsc.VectorSubcoreMesh(core_axis_name="c", subcore_axis_name="s")


@functools.partial(
    pl.kernel,
    out_type=jax.ShapeDtypeStruct((3, _NC, _NPAD, _W16), jnp.float32),
    mesh=_mesh(),
    scratch_types=[
        pltpu.VMEM((_NCHD, _CKD), jnp.int32),
        pltpu.VMEM((_CKD, _W16), jnp.float32),
        pltpu.VMEM_SHARED((_NPAD, _W16), jnp.float32),
        pltpu.SemaphoreType.DMA,
        pltpu.SemaphoreType.DMA,
        pltpu.SemaphoreType.DMA,
        pltpu.SemaphoreType.DMA,
    ],
)
def _degk(r3_ui, r3_ia, r3_ua, z16, onerows, out, ivd, ones_v, acc,
          dsem0, dsem1, dsem2, dsem3):
    c = lax.axis_index("c")
    s = lax.axis_index("s")
    w = c * _NS + s
    sems = (dsem0, dsem1, dsem2, dsem3)
    pltpu.sync_copy(onerows, ones_v)
    for g, rr in enumerate((r3_ui, r3_ia, r3_ua)):
        pltpu.sync_copy(z16, acc.at[pl.ds(s * _SL, _SL)])
        pltpu.sync_copy(rr.at[w], ivd)
        plsc.subcore_barrier()

        @pl.loop(0, _NCHD, step=4)
        def _chunk(k):
            for b in range(4):
                pltpu.async_copy(ones_v, acc.at[ivd.at[k + b]], sems[b],
                                 add=True)
            for b in range(4):
                pltpu.make_async_copy(ones_v, acc.at[ivd.at[k + b]],
                                      sems[b]).wait()

        plsc.subcore_barrier()
        pltpu.sync_copy(acc.at[pl.ds(s * _SL, _SL)],
                        out.at[g, c, pl.ds(s * _SL, _SL)])
        plsc.subcore_barrier()


@functools.partial(
    pl.kernel,
    out_type=jax.ShapeDtypeStruct((_NC, _NPAD, _D), jnp.float32),
    mesh=_mesh(),
    scratch_types=[
        pltpu.VMEM((_IBK, 2, _CK), jnp.int32),
        pltpu.VMEM((_NBUF, _CK, _D), jnp.float32),
        pltpu.VMEM_SHARED((_NPAD, _D), jnp.float32),
        pltpu.SemaphoreType.DMA,
        pltpu.SemaphoreType.DMA,
        pltpu.SemaphoreType.DMA,
        pltpu.SemaphoreType.DMA,
        pltpu.SemaphoreType.DMA,
        pltpu.SemaphoreType.DMA,
        pltpu.SemaphoreType.DMA,
        pltpu.SemaphoreType.DMA,
    ],
)
def _adjsum(table, rc3, zrows, out, iv, rbuf, acc,
            g0, g1, g2, g3, s0, s1, s2, s3):
    c = lax.axis_index("c")
    s = lax.axis_index("s")
    w = c * _NS + s
    pltpu.sync_copy(zrows, acc.at[pl.ds(s * _SL, _SL)])
    plsc.subcore_barrier()
    gsems = (g0, g1, g2, g3)
    ssems = (s0, s1, s2, s3)

    def gstart(k, b):
        pltpu.async_copy(table.at[iv.at[k, 0]], rbuf.at[b], gsems[b])

    def gwait(k, b):
        pltpu.make_async_copy(table.at[iv.at[k, 0]], rbuf.at[b],
                              gsems[b]).wait()

    def sstart(k, b):
        pltpu.async_copy(rbuf.at[b], acc.at[iv.at[k, 1]], ssems[b], add=True)

    def swait(k, b):
        pltpu.make_async_copy(rbuf.at[b], acc.at[iv.at[k, 1]],
                              ssems[b]).wait()

    def ring(chunk_base, nch):
        @pl.loop(0, nch // _IBK)
        def _blk(j):
            pltpu.sync_copy(rc3.at[pl.ds(chunk_base + j * _IBK, _IBK)], iv)
            for b in range(_NBUF):
                gstart(b, b)

            @pl.loop(0, _IBK, step=_NBUF)
            def _grp(k):
                for b in range(_NBUF):
                    gwait(k + b, b)
                    sstart(k + b, b)
                for b in range(_NBUF):
                    @pl.when(k + _NBUF + b < _IBK)
                    def _(b=b):
                        swait(k + b, b)
                        gstart(k + _NBUF + b, b)

            for b in range(_NBUF):
                swait(_IBK - _NBUF + b, b)

    @pl.when(c == 0)
    def _():
        ring(s * _CH0, _CH0)

    @pl.when(c == 1)
    def _():
        ring(16 * _CH0 + s * _CH1, _CH1)

    plsc.subcore_barrier()
    pltpu.sync_copy(acc.at[pl.ds(s * _SL, _SL)],
                    out.at[c, pl.ds(s * _SL, _SL)])


@functools.partial(
    pl.kernel,
    out_type=jax.ShapeDtypeStruct((5, 2 * _B, _D), jnp.float32),
    mesh=_mesh(),
    scratch_types=[
        pltpu.VMEM((128,), jnp.int32),
        pltpu.VMEM((128, _D), jnp.float32),
        pltpu.SemaphoreType.DMA,
    ],
)
def _gather5(ui_rep, ua_rep, ia_rep, hat, utd, itd, itmd, out, iv, rbuf, sem):
    c = lax.axis_index("c")
    s = lax.axis_index("s")
    w = c * _NS + s
    rpw = (2 * _B) // 32
    tabs = (ui_rep, ui_rep, ua_rep, ia_rep, hat)
    idxs = (utd, itd, utd, itmd, utd)
    for t in range(5):
        for j in range(rpw // 128):
            base = w * rpw + j * 128
            pltpu.sync_copy(idxs[t].at[pl.ds(base, 128)], iv)
            pltpu.async_copy(tabs[t].at[iv], rbuf, sem).wait()
            pltpu.sync_copy(rbuf, out.at[t, pl.ds(base, 128)])


def _disp_body(dp, idp, iap, uap, disb, rcpb, xs):
    d = dp[...]
    deg16 = d[:, 0] + d[:, 1]
    deg = jnp.broadcast_to(deg16[:, :, 0:1], (3, _BR, _D))
    dis = jnp.where(deg > 0.0, lax.rsqrt(deg), 0.0)
    disb[...] = dis
    rcpb[...] = 1.0 / jnp.maximum(deg[2], 1.0)
    xs[0] = dis[0] * idp[...]
    xs[1] = dis[1] * iap[...]
    xs[2] = dis[2] * uap[...]


_disp = pl.pallas_call(
    _disp_body,
    grid=(_GRID,),
    in_specs=[
        pl.BlockSpec((3, _NC, _BR, _W16), lambda i: (0, 0, i, 0)),
        pl.BlockSpec((_BR, _D), lambda i: (i, 0)),
        pl.BlockSpec((_BR, _D), lambda i: (i, 0)),
        pl.BlockSpec((_BR, _D), lambda i: (i, 0)),
    ],
    out_specs=[
        pl.BlockSpec((3, _BR, _D), lambda i: (0, i, 0)),
        pl.BlockSpec((_BR, _D), lambda i: (i, 0)),
        pl.BlockSpec((3, _BR, _D), lambda i: (0, i, 0)),
    ],
    out_shape=[
        jax.ShapeDtypeStruct((3, _NPAD, _D), jnp.float32),
        jax.ShapeDtypeStruct((_NPAD, _D), jnp.float32),
        jax.ShapeDtypeStruct((3, _NPAD, _D), jnp.float32),
    ],
)


def _merge_specs(n_in, n_out):
    ins = [pl.BlockSpec((_NC, _BR, _D), lambda i: (0, i, 0))]
    ins += [pl.BlockSpec((_BR, _D), lambda i: (i, 0)) for _ in range(n_in - 1)]
    outs = [pl.BlockSpec((_BR, _D), lambda i: (i, 0)) for _ in range(n_out)]
    shp = [jax.ShapeDtypeStruct((_NPAD, _D), jnp.float32) for _ in range(n_out)]
    return dict(grid=(_GRID,), in_specs=ins, out_specs=outs, out_shape=shp)


def _merge_mid_body(p, db, b, accn, xsn):
    t = db[...] * (p[0] + p[1])
    accn[...] = b[...] + t
    xsn[...] = db[...] * t


_merge_mid = pl.pallas_call(_merge_mid_body, **_merge_specs(3, 2))


def _make_merge_fin(scale):
    def body(p, db, b, rep):
        rep[...] = (b[...] + db[...] * (p[0] + p[1])) * scale
    return pl.pallas_call(body, **_merge_specs(3, 1))


_merge_fin4 = _make_merge_fin(0.25)
_merge_fin2 = _make_merge_fin(0.5)


def _merge_mean_body(p, rcp, hat):
    hat[...] = rcp[...] * (p[0] + p[1])


_merge_mean = pl.pallas_call(_merge_mean_body, **_merge_specs(2, 1))


def _loss_body(p_ref, n_ref, out_ref, acc_ref):
    i = pl.program_id(0)
    p = p_ref[...]
    n = n_ref[...]

    def dots(x):
        cf = jnp.sum(x[0] * x[1], axis=-1, keepdims=True)
        kg = jnp.sum(x[2] * x[3], axis=-1, keepdims=True)
        ha = jnp.sum(x[4] * x[3], axis=-1, keepdims=True)
        return cf, kg, ha

    cfp, kgp, hap = dots(p)
    cfn, kgn, han = dots(n)
    sp_ = jax.nn.sigmoid(hap)
    sn_ = jax.nn.sigmoid(han)
    z = (cfn + kgn * sn_) - (cfp + kgp * sp_)
    l1 = jnp.maximum(z, 0.0) + jnp.log(1.0 + jnp.exp(-jnp.abs(z)))
    l2 = jnp.maximum(sp_ - sn_ - _MARGIN, 0.0)

    @pl.when(i == 0)
    def _():
        acc_ref[0] = 0.0
        acc_ref[1] = 0.0

    acc_ref[0] += jnp.sum(l1)
    acc_ref[1] += jnp.sum(l2)

    @pl.when(i == pl.num_programs(0) - 1)
    def _():
        out_ref[...] = jnp.full((1, 1),
                                acc_ref[0] / _B + _ALPHA * (acc_ref[1] / _B))


_loss_call = pl.pallas_call(
    _loss_body,
    grid=(_B // _BRL,),
    in_specs=[
        pl.BlockSpec((5, _BRL, _D), lambda i: (0, i, 0)),
        pl.BlockSpec((5, _BRL, _D), lambda i: (0, i + _B // _BRL, 0)),
    ],
    out_specs=pl.BlockSpec((1, 1), lambda i: (0, 0)),
    out_shape=jax.ShapeDtypeStruct((1, 1), jnp.float32),
    scratch_shapes=[pltpu.SMEM((2,), jnp.float32)],
)


def _reg_body(idr, upr, ipr, out):
    out[...] = jnp.full((1, 1), jnp.sum(idr[...] ** 2) / (_N * _D)
                        + jnp.sum(upr[...] ** 2) / (_NU * _D)
                        + jnp.sum(ipr[...] ** 2) / (_NI * _D))


_reg_call = pl.pallas_call(
    _reg_body,
    grid=(1,),
    in_specs=[
        pl.BlockSpec((_N, _D), lambda i: (0, 0)),
        pl.BlockSpec((_NU, _D), lambda i: (0, 0)),
        pl.BlockSpec((_NI, _D), lambda i: (0, 0)),
    ],
    out_specs=pl.BlockSpec((1, 1), lambda i: (0, 0)),
    out_shape=jax.ShapeDtypeStruct((1, 1), jnp.float32),
)


def _pad_edges(ei):
    ei = ei.astype(jnp.int32)
    pad = jnp.full((_EPAD - _E,), _N, jnp.int32)
    rows = jnp.concatenate([ei[0], pad]).reshape(_TOTCH, 1, _CK)
    cols = jnp.concatenate([ei[1], pad]).reshape(_TOTCH, 1, _CK)
    rc3 = jnp.concatenate([rows, cols], axis=1)
    return rc3, rows.reshape(32, _NCHD, _CKD)


def kernel(id_embedding, user_pre, item_pre, attribute, user_tensor,
           item_tensor, ui_edge_index, ia_edge_index, ua_edge_index):
    rowpad = ((0, _NPAD - _N), (0, 0))
    idp = jnp.pad(id_embedding, rowpad)
    ia0p = jnp.pad(jnp.concatenate([item_pre, attribute], axis=0), rowpad)
    ua0p = jnp.pad(jnp.concatenate([user_pre, attribute], axis=0), rowpad)
    rc_ui, r3_ui = _pad_edges(ui_edge_index)
    rc_ia, r3_ia = _pad_edges(ia_edge_index)
    rc_ua, r3_ua = _pad_edges(ua_edge_index)
    zrows = jnp.zeros((_SL, _D), jnp.float32)
    z16 = jnp.zeros((_SL, _W16), jnp.float32)
    onerows = jnp.ones((_CKD, _W16), jnp.float32)

    degp = _degk(r3_ui, r3_ia, r3_ua, z16, onerows)
    disb, rcpb, xs0 = _disp(degp, idp, ia0p, ua0p)

    p = _adjsum(xs0[0], rc_ui, zrows)
    acc1, xs = _merge_mid(p, disb[0], idp)
    p = _adjsum(xs, rc_ui, zrows)
    acc2, xs = _merge_mid(p, disb[0], acc1)
    p = _adjsum(xs, rc_ui, zrows)
    ui_rep, = _merge_fin4(p, disb[0], acc2)

    p = _adjsum(xs0[1], rc_ia, zrows)
    ia_rep, = _merge_fin2(p, disb[1], ia0p)

    p = _adjsum(xs0[2], rc_ua, zrows)
    ua_rep, = _merge_fin2(p, disb[2], ua0p)

    p = _adjsum(ua0p, rc_ua, zrows)
    hat, = _merge_mean(p, rcpb)

    ut = user_tensor.astype(jnp.int32)
    it = item_tensor.astype(jnp.int32)
    utd = jnp.concatenate([ut[:, 0], ut[:, 1]])
    itd = jnp.concatenate([it[:, 0], it[:, 1]])

    g5 = _gather5(ui_rep, ua_rep, ia_rep, hat, utd, itd, itd - _NU)
    loss = _loss_call(g5, g5)
    reg = _reg_call(id_embedding, user_pre, item_pre)
    return loss[0, 0], reg[0, 0]
